# baseline (speedup 1.0000x reference)
import math

import jax
import jax.numpy as jnp
from jax import lax
from jax.experimental import pallas as pl
from jax.experimental.pallas import tpu as pltpu
from jax.experimental.pallas import tpu_sc as plsc

_EMB = 64
_SCALE = math.sqrt(_EMB)
_NC, _NS = 2, 16
_NW = _NC * _NS
_BB = 128
_L = 200
_B = 4096
_SP = _BB + 1

_V = 1000000
_PW = 256
_NPF = _V // _PW
_KF = _NPF // _NW
_NEX = _NPF - _KF * _NW
_TAIL = _V - _NPF * _PW
_RP = 2 * _EMB


def _tt_transpose(buf, rows2, b, width):
    iot = lax.iota(jnp.int32, 16)
    zero = jnp.zeros((16,), jnp.int32)

    def seg(s, carry):
        cvec = iot + s * 16
        for f0 in range(0, _EMB, 8):
            vs = [buf[b, f0 + i, pl.ds(s * 16, 16)] for i in range(8)]
            for i in range(8):
                plsc.store_scatter(rows2.at[b], [cvec, zero + (f0 + i)], vs[i])
        return carry

    lax.fori_loop(0, width // 16, seg, 0)


def _tt_body(tab_hbm, tail_hbm, lin_hbm, buf, rows2, gsem0, gsem1, wsem0,
             wsem1):
    w = lax.axis_index("s") * _NC + lax.axis_index("c")
    gsems = (gsem0, gsem1)
    wsems = (wsem0, wsem1)

    def fetch(j, b):
        pltpu.async_copy(
            tab_hbm.at[:, pl.ds(j * _PW, _PW)], buf.at[b], gsems[b])

    def wait_fetch(b):
        pltpu.make_async_copy(
            tab_hbm.at[:, pl.ds(0, _PW)], buf.at[b], gsems[b]).wait()

    def store(j, b):
        pltpu.async_copy(
            rows2.at[b, :, pl.ds(0, 2 * _EMB)],
            lin_hbm.at[pl.ds(j * _PW, _PW)], wsems[b])

    def wait_store(b):
        pltpu.make_async_copy(
            rows2.at[b, :, pl.ds(0, 2 * _EMB)],
            lin_hbm.at[pl.ds(0, _PW)], wsems[b]).wait()

    for b in range(2):
        fetch(w + _NW * b, b)
    for b in range(2):
        wait_fetch(b)
        _tt_transpose(buf, rows2, b, _PW)
        store(w + _NW * b, b)
        fetch(w + _NW * (b + 2), b)

    def body(p, carry):
        for b in range(2):
            k = 2 * p + b
            wait_fetch(b)
            wait_store(b)
            _tt_transpose(buf, rows2, b, _PW)
            store(w + _NW * k, b)

            @pl.when(p < _KF // 2 - 1)
            def _():
                fetch(w + _NW * (k + 2), b)
        return carry

    lax.fori_loop(1, _KF // 2, body, 0)
    for b in range(2):
        wait_store(b)

    @pl.when(w < _NEX)
    def _():
        j = _KF * _NW + w
        pltpu.sync_copy(tab_hbm.at[:, pl.ds(j * _PW, _PW)], buf.at[0])
        _tt_transpose(buf, rows2, 0, _PW)
        pltpu.sync_copy(
            rows2.at[0, :, pl.ds(0, 2 * _EMB)],
            lin_hbm.at[pl.ds(j * _PW, _PW)])

    @pl.when(w == _NEX)
    def _():
        pltpu.sync_copy(
            tail_hbm, rows2.at[1, pl.ds(0, _TAIL), pl.ds(0, 2 * _EMB)])
        pltpu.sync_copy(
            rows2.at[1, pl.ds(0, _TAIL), pl.ds(0, 2 * _EMB)],
            lin_hbm.at[pl.ds(_NPF * _PW, _TAIL)])


def _transpose_unit(rows, stage, b):
    iot = lax.iota(jnp.int32, 16)
    avecs = [iot // 8 + (f0 // 8) for f0 in range(0, _EMB, 16)]
    rvecs = [iot % 8 for _ in range(0, _EMB, 16)]
    zero = jnp.zeros((16,), jnp.int32)

    def col(c, carry):
        cvec = zero + c
        vs = [rows[b, c, pl.ds(f0, 16)] * _SCALE
              for f0 in range(0, _EMB, 16)]
        for k in range(_EMB // 16):
            plsc.store_scatter(stage.at[b], [avecs[k], rvecs[k], cvec], vs[k])
        return carry

    lax.fori_loop(0, _BB, col, 0, unroll=4)


def _sc_body(tok_hbm, table_hbm, out_hbm, idx_v, rows, stage, gsem0, gsem1,
             wsem0, wsem1):
    w = lax.axis_index("s") * _NC + lax.axis_index("c")
    gsems = (gsem0, gsem1)
    wsems = (wsem0, wsem1)

    pltpu.sync_copy(tok_hbm.at[:, pl.ds(w * _BB, _BB)], idx_v)

    def dbl(l, carry):
        for k in range(_BB // 16):
            sl = pl.ds(k * 16, 16)
            idx_v[l, sl] = idx_v[l, sl] * 2
        return carry

    lax.fori_loop(0, _L, dbl, 0, unroll=4)

    def gather(l, b):
        pltpu.async_copy(table_hbm.at[idx_v.at[l]], rows.at[b], gsems[b])

    def wait_gather(b):
        pltpu.make_async_copy(
            table_hbm.at[idx_v.at[0]], rows.at[b], gsems[b]).wait()

    def store(l, b):
        pltpu.async_copy(
            stage.at[b, :, :, pl.ds(0, _BB)], out_hbm.at[l, :, w], wsems[b])

    def wait_store(b):
        pltpu.make_async_copy(
            stage.at[b, :, :, pl.ds(0, _BB)], out_hbm.at[0, :, w],
            wsems[b]).wait()

    for b in range(2):
        gather(b, b)
    for b in range(2):
        wait_gather(b)
        _transpose_unit(rows, stage, b)
        store(b, b)
        gather(b + 2, b)

    def pair(p, carry):
        for b in range(2):
            l = 2 * p + b
            wait_gather(b)
            wait_store(b)
            _transpose_unit(rows, stage, b)
            store(l, b)

            @pl.when(p < _L // 2 - 1)
            def _():
                gather(l + 2, b)
        return carry

    lax.fori_loop(1, _L // 2, pair, 0)

    for b in range(2):
        wait_store(b)


def kernel(tokens, table):
    tok_t = tokens.T.astype(jnp.int32)
    tab_t = table.T
    tail_rows = jnp.pad(table[_NPF * _PW:], ((0, 0), (0, _EMB)))
    mesh = plsc.VectorSubcoreMesh(
        core_axis_name="c", subcore_axis_name="s",
        num_cores=_NC, num_subcores=_NS,
    )

    relayout = pl.kernel(
        _tt_body,
        out_type=jax.ShapeDtypeStruct((_V, 2 * _EMB), jnp.float32),
        mesh=mesh,
        scratch_types=[
            pltpu.VMEM((2, _EMB, _PW), jnp.float32),
            pltpu.VMEM((2, _PW, _RP), jnp.float32),
            pltpu.SemaphoreType.DMA,
            pltpu.SemaphoreType.DMA,
            pltpu.SemaphoreType.DMA,
            pltpu.SemaphoreType.DMA,
        ],
        compiler_params=pltpu.CompilerParams(
            use_tc_tiling_on_sc=True, needs_layout_passes=False),
    )
    lin = relayout(tab_t, tail_rows).reshape(2 * _V, _EMB)

    run = pl.kernel(
        _sc_body,
        out_type=jax.ShapeDtypeStruct(
            (_L, _EMB // 8, _B // _BB, 8, _BB), jnp.float32),
        mesh=mesh,
        scratch_types=[
            pltpu.VMEM((_L, _BB), jnp.int32),
            pltpu.VMEM((2, _BB, _EMB), jnp.float32),
            pltpu.VMEM((2, _EMB // 8, 8, _SP), jnp.float32),
            pltpu.SemaphoreType.DMA,
            pltpu.SemaphoreType.DMA,
            pltpu.SemaphoreType.DMA,
            pltpu.SemaphoreType.DMA,
        ],
        compiler_params=pltpu.CompilerParams(
            use_tc_tiling_on_sc=False, needs_layout_passes=False),
    )
    out5 = run(tok_t, lin)
    return out5.transpose(2, 4, 0, 1, 3).reshape(_B, _L, _EMB)


# speedup vs baseline: 1.4934x; 1.4934x over previous
import math

import jax
import jax.numpy as jnp
from jax import lax
from jax.experimental import pallas as pl
from jax.experimental.pallas import tpu as pltpu
from jax.experimental.pallas import tpu_sc as plsc

_EMB = 64
_SCALE = math.sqrt(_EMB)
_NC, _NS = 2, 16
_NW = _NC * _NS
_BB = 128
_L = 200
_B = 4096
_SP = _BB + 1

_V = 1000000
_PW = 256
_NPF = _V // _PW
_KF = _NPF // _NW
_NEX = _NPF - _KF * _NW
_TAIL = _V - _NPF * _PW
_RP = 2 * _EMB


def _tt_transpose(buf, rows2, b, width):
    iot = lax.iota(jnp.int32, 16)
    rots = [(iot + k) % 16 for k in range(16)]

    def seg(s, carry):
        for fb in range(_EMB // 16):
            fvec = iot + fb * 16
            for k in range(16):
                cvec = rots[k] + s * 16
                v = plsc.load_gather(buf.at[b], [fvec, cvec])
                plsc.store_scatter(rows2.at[b], [cvec, fvec], v)
        return carry

    lax.fori_loop(0, width // 16, seg, 0)


def _tt_body(tab_hbm, tail_hbm, lin_hbm, buf, rows2, gsem0, gsem1, wsem0,
             wsem1):
    w = lax.axis_index("s") * _NC + lax.axis_index("c")
    gsems = (gsem0, gsem1)
    wsems = (wsem0, wsem1)

    def fetch(j, b):
        pltpu.async_copy(
            tab_hbm.at[:, pl.ds(j * _PW, _PW)], buf.at[b], gsems[b])

    def wait_fetch(b):
        pltpu.make_async_copy(
            tab_hbm.at[:, pl.ds(0, _PW)], buf.at[b], gsems[b]).wait()

    def store(j, b):
        pltpu.async_copy(
            rows2.at[b, :, pl.ds(0, 2 * _EMB)],
            lin_hbm.at[pl.ds(j * _PW, _PW)], wsems[b])

    def wait_store(b):
        pltpu.make_async_copy(
            rows2.at[b, :, pl.ds(0, 2 * _EMB)],
            lin_hbm.at[pl.ds(0, _PW)], wsems[b]).wait()

    for b in range(2):
        fetch(w + _NW * b, b)
    for b in range(2):
        wait_fetch(b)
        _tt_transpose(buf, rows2, b, _PW)
        store(w + _NW * b, b)
        fetch(w + _NW * (b + 2), b)

    def body(p, carry):
        for b in range(2):
            k = 2 * p + b
            wait_fetch(b)
            wait_store(b)
            _tt_transpose(buf, rows2, b, _PW)
            store(w + _NW * k, b)

            @pl.when(p < _KF // 2 - 1)
            def _():
                fetch(w + _NW * (k + 2), b)
        return carry

    lax.fori_loop(1, _KF // 2, body, 0)
    for b in range(2):
        wait_store(b)

    @pl.when(w < _NEX)
    def _():
        j = _KF * _NW + w
        pltpu.sync_copy(tab_hbm.at[:, pl.ds(j * _PW, _PW)], buf.at[0])
        _tt_transpose(buf, rows2, 0, _PW)
        pltpu.sync_copy(
            rows2.at[0, :, pl.ds(0, 2 * _EMB)],
            lin_hbm.at[pl.ds(j * _PW, _PW)])

    @pl.when(w == _NEX)
    def _():
        pltpu.sync_copy(
            tail_hbm, rows2.at[1, pl.ds(0, _TAIL), pl.ds(0, 2 * _EMB)])
        pltpu.sync_copy(
            rows2.at[1, pl.ds(0, _TAIL), pl.ds(0, 2 * _EMB)],
            lin_hbm.at[pl.ds(_NPF * _PW, _TAIL)])


def _transpose_unit(rows, stage, b):
    iot = lax.iota(jnp.int32, 16)
    avecs = [iot // 8 + (f0 // 8) for f0 in range(0, _EMB, 16)]
    rvecs = [iot % 8 for _ in range(0, _EMB, 16)]
    zero = jnp.zeros((16,), jnp.int32)

    def col(c, carry):
        cvec = zero + c
        vs = [rows[b, c, pl.ds(f0, 16)] * _SCALE
              for f0 in range(0, _EMB, 16)]
        for k in range(_EMB // 16):
            plsc.store_scatter(stage.at[b], [avecs[k], rvecs[k], cvec], vs[k])
        return carry

    lax.fori_loop(0, _BB, col, 0, unroll=4)


def _sc_body(tok_hbm, table_hbm, out_hbm, idx_v, rows, stage, gsem0, gsem1,
             wsem0, wsem1):
    w = lax.axis_index("s") * _NC + lax.axis_index("c")
    gsems = (gsem0, gsem1)
    wsems = (wsem0, wsem1)

    pltpu.sync_copy(tok_hbm.at[:, pl.ds(w * _BB, _BB)], idx_v)

    def dbl(l, carry):
        for k in range(_BB // 16):
            sl = pl.ds(k * 16, 16)
            idx_v[l, sl] = idx_v[l, sl] * 2
        return carry

    lax.fori_loop(0, _L, dbl, 0, unroll=4)

    def gather(l, b):
        pltpu.async_copy(table_hbm.at[idx_v.at[l]], rows.at[b], gsems[b])

    def wait_gather(b):
        pltpu.make_async_copy(
            table_hbm.at[idx_v.at[0]], rows.at[b], gsems[b]).wait()

    def store(l, b):
        pltpu.async_copy(
            stage.at[b, :, :, pl.ds(0, _BB)], out_hbm.at[l, :, w], wsems[b])

    def wait_store(b):
        pltpu.make_async_copy(
            stage.at[b, :, :, pl.ds(0, _BB)], out_hbm.at[0, :, w],
            wsems[b]).wait()

    for b in range(2):
        gather(b, b)
    for b in range(2):
        wait_gather(b)
        _transpose_unit(rows, stage, b)
        store(b, b)
        gather(b + 2, b)

    def pair(p, carry):
        for b in range(2):
            l = 2 * p + b
            wait_gather(b)
            wait_store(b)
            _transpose_unit(rows, stage, b)
            store(l, b)

            @pl.when(p < _L // 2 - 1)
            def _():
                gather(l + 2, b)
        return carry

    lax.fori_loop(1, _L // 2, pair, 0)

    for b in range(2):
        wait_store(b)


def kernel(tokens, table):
    tok_t = tokens.T.astype(jnp.int32)
    tab_t = table.T
    tail_rows = jnp.pad(table[_NPF * _PW:], ((0, 0), (0, _EMB)))
    mesh = plsc.VectorSubcoreMesh(
        core_axis_name="c", subcore_axis_name="s",
        num_cores=_NC, num_subcores=_NS,
    )

    relayout = pl.kernel(
        _tt_body,
        out_type=jax.ShapeDtypeStruct((_V, 2 * _EMB), jnp.float32),
        mesh=mesh,
        scratch_types=[
            pltpu.VMEM((2, _EMB, _PW), jnp.float32),
            pltpu.VMEM((2, _PW, _RP), jnp.float32),
            pltpu.SemaphoreType.DMA,
            pltpu.SemaphoreType.DMA,
            pltpu.SemaphoreType.DMA,
            pltpu.SemaphoreType.DMA,
        ],
        compiler_params=pltpu.CompilerParams(
            use_tc_tiling_on_sc=True, needs_layout_passes=False),
    )
    lin = relayout(tab_t, tail_rows).reshape(2 * _V, _EMB)

    run = pl.kernel(
        _sc_body,
        out_type=jax.ShapeDtypeStruct(
            (_L, _EMB // 8, _B // _BB, 8, _BB), jnp.float32),
        mesh=mesh,
        scratch_types=[
            pltpu.VMEM((_L, _BB), jnp.int32),
            pltpu.VMEM((2, _BB, _EMB), jnp.float32),
            pltpu.VMEM((2, _EMB // 8, 8, _SP), jnp.float32),
            pltpu.SemaphoreType.DMA,
            pltpu.SemaphoreType.DMA,
            pltpu.SemaphoreType.DMA,
            pltpu.SemaphoreType.DMA,
        ],
        compiler_params=pltpu.CompilerParams(
            use_tc_tiling_on_sc=False, needs_layout_passes=False),
    )
    out5 = run(tok_t, lin)
    return out5.transpose(2, 4, 0, 1, 3).reshape(_B, _L, _EMB)


# speedup vs baseline: 2.0309x; 1.3600x over previous
import math

import jax
import jax.numpy as jnp
from jax import lax
from jax.experimental import pallas as pl
from jax.experimental.pallas import tpu as pltpu
from jax.experimental.pallas import tpu_sc as plsc

_EMB = 64
_SCALE = math.sqrt(_EMB)
_NC, _NS = 2, 16
_NW = _NC * _NS
_BB = 128
_L = 200
_B = 4096
_SP = _BB + 1

_V = 1000000
_PW = 256
_NPF = _V // _PW
_KF = _NPF // _NW
_NEX = _NPF - _KF * _NW
_TAIL = _V - _NPF * _PW
_RP = 2 * _EMB


def _tt_transpose(buf, rows2, b, width):
    iot = lax.iota(jnp.int32, 16)
    rots = [(iot + k) % 16 for k in range(16)]

    def seg(s, carry):
        for fb in range(_EMB // 16):
            fvec = iot + fb * 16
            for k0 in range(0, 16, 8):
                cvecs = [rots[k0 + k] + s * 16 for k in range(8)]
                vs = [plsc.load_gather(buf.at[b], [fvec, cvecs[k]])
                      for k in range(8)]
                for k in range(8):
                    plsc.store_scatter(rows2.at[b], [cvecs[k], fvec], vs[k])
        return carry

    lax.fori_loop(0, width // 16, seg, 0)


def _tt_body(tab_hbm, tail_hbm, lin_hbm, buf, rows2, gsem0, gsem1, wsem0,
             wsem1):
    w = lax.axis_index("s") * _NC + lax.axis_index("c")
    gsems = (gsem0, gsem1)
    wsems = (wsem0, wsem1)

    def fetch(j, b):
        pltpu.async_copy(
            tab_hbm.at[:, pl.ds(j * _PW, _PW)], buf.at[b], gsems[b])

    def wait_fetch(b):
        pltpu.make_async_copy(
            tab_hbm.at[:, pl.ds(0, _PW)], buf.at[b], gsems[b]).wait()

    def store(j, b):
        pltpu.async_copy(
            rows2.at[b, :, pl.ds(0, 2 * _EMB)],
            lin_hbm.at[pl.ds(j * _PW, _PW)], wsems[b])

    def wait_store(b):
        pltpu.make_async_copy(
            rows2.at[b, :, pl.ds(0, 2 * _EMB)],
            lin_hbm.at[pl.ds(0, _PW)], wsems[b]).wait()

    for b in range(2):
        fetch(w + _NW * b, b)
    for b in range(2):
        wait_fetch(b)
        _tt_transpose(buf, rows2, b, _PW)
        store(w + _NW * b, b)
        fetch(w + _NW * (b + 2), b)

    def body(p, carry):
        for b in range(2):
            k = 2 * p + b
            wait_fetch(b)
            wait_store(b)
            _tt_transpose(buf, rows2, b, _PW)
            store(w + _NW * k, b)

            @pl.when(p < _KF // 2 - 1)
            def _():
                fetch(w + _NW * (k + 2), b)
        return carry

    lax.fori_loop(1, _KF // 2, body, 0)
    for b in range(2):
        wait_store(b)

    @pl.when(w < _NEX)
    def _():
        j = _KF * _NW + w
        pltpu.sync_copy(tab_hbm.at[:, pl.ds(j * _PW, _PW)], buf.at[0])
        _tt_transpose(buf, rows2, 0, _PW)
        pltpu.sync_copy(
            rows2.at[0, :, pl.ds(0, 2 * _EMB)],
            lin_hbm.at[pl.ds(j * _PW, _PW)])

    @pl.when(w == _NEX)
    def _():
        pltpu.sync_copy(
            tail_hbm, rows2.at[1, pl.ds(0, _TAIL), pl.ds(0, 2 * _EMB)])
        pltpu.sync_copy(
            rows2.at[1, pl.ds(0, _TAIL), pl.ds(0, 2 * _EMB)],
            lin_hbm.at[pl.ds(_NPF * _PW, _TAIL)])


def _transpose_unit(rows, stage, b):
    iot = lax.iota(jnp.int32, 16)
    avecs = [iot // 8 + (f0 // 8) for f0 in range(0, _EMB, 16)]
    rvecs = [iot % 8 for _ in range(0, _EMB, 16)]
    zero = jnp.zeros((16,), jnp.int32)

    def col(c, carry):
        cvec = zero + c
        vs = [rows[b, c, pl.ds(f0, 16)] * _SCALE
              for f0 in range(0, _EMB, 16)]
        for k in range(_EMB // 16):
            plsc.store_scatter(stage.at[b], [avecs[k], rvecs[k], cvec], vs[k])
        return carry

    lax.fori_loop(0, _BB, col, 0, unroll=4)


def _sc_body(tok_hbm, table_hbm, out_hbm, idx_v, rows, stage, gsem0, gsem1,
             wsem0, wsem1):
    w = lax.axis_index("s") * _NC + lax.axis_index("c")
    gsems = (gsem0, gsem1)
    wsems = (wsem0, wsem1)

    pltpu.sync_copy(tok_hbm.at[:, pl.ds(w * _BB, _BB)], idx_v)

    def dbl(l, carry):
        for k in range(_BB // 16):
            sl = pl.ds(k * 16, 16)
            idx_v[l, sl] = idx_v[l, sl] * 2
        return carry

    lax.fori_loop(0, _L, dbl, 0, unroll=4)

    def gather(l, b):
        pltpu.async_copy(table_hbm.at[idx_v.at[l]], rows.at[b], gsems[b])

    def wait_gather(b):
        pltpu.make_async_copy(
            table_hbm.at[idx_v.at[0]], rows.at[b], gsems[b]).wait()

    def store(l, b):
        pltpu.async_copy(
            stage.at[b, :, :, pl.ds(0, _BB)], out_hbm.at[l, :, w], wsems[b])

    def wait_store(b):
        pltpu.make_async_copy(
            stage.at[b, :, :, pl.ds(0, _BB)], out_hbm.at[0, :, w],
            wsems[b]).wait()

    for b in range(2):
        gather(b, b)
    for b in range(2):
        wait_gather(b)
        _transpose_unit(rows, stage, b)
        store(b, b)
        gather(b + 2, b)

    def pair(p, carry):
        for b in range(2):
            l = 2 * p + b
            wait_gather(b)
            wait_store(b)
            _transpose_unit(rows, stage, b)
            store(l, b)

            @pl.when(p < _L // 2 - 1)
            def _():
                gather(l + 2, b)
        return carry

    lax.fori_loop(1, _L // 2, pair, 0)

    for b in range(2):
        wait_store(b)


def kernel(tokens, table):
    tok_t = tokens.T.astype(jnp.int32)
    tab_t = table.T
    tail_rows = jnp.pad(table[_NPF * _PW:], ((0, 0), (0, _EMB)))
    mesh = plsc.VectorSubcoreMesh(
        core_axis_name="c", subcore_axis_name="s",
        num_cores=_NC, num_subcores=_NS,
    )

    relayout = pl.kernel(
        _tt_body,
        out_type=jax.ShapeDtypeStruct((_V, 2 * _EMB), jnp.float32),
        mesh=mesh,
        scratch_types=[
            pltpu.VMEM((2, _EMB, _PW), jnp.float32),
            pltpu.VMEM((2, _PW, _RP), jnp.float32),
            pltpu.SemaphoreType.DMA,
            pltpu.SemaphoreType.DMA,
            pltpu.SemaphoreType.DMA,
            pltpu.SemaphoreType.DMA,
        ],
        compiler_params=pltpu.CompilerParams(
            use_tc_tiling_on_sc=True, needs_layout_passes=False),
    )
    lin = relayout(tab_t, tail_rows).reshape(2 * _V, _EMB)

    run = pl.kernel(
        _sc_body,
        out_type=jax.ShapeDtypeStruct(
            (_L, _EMB // 8, _B // _BB, 8, _BB), jnp.float32),
        mesh=mesh,
        scratch_types=[
            pltpu.VMEM((_L, _BB), jnp.int32),
            pltpu.VMEM((2, _BB, _EMB), jnp.float32),
            pltpu.VMEM((2, _EMB // 8, 8, _SP), jnp.float32),
            pltpu.SemaphoreType.DMA,
            pltpu.SemaphoreType.DMA,
            pltpu.SemaphoreType.DMA,
            pltpu.SemaphoreType.DMA,
        ],
        compiler_params=pltpu.CompilerParams(
            use_tc_tiling_on_sc=False, needs_layout_passes=False),
    )
    out5 = run(tok_t, lin)
    return out5.transpose(2, 4, 0, 1, 3).reshape(_B, _L, _EMB)
